# baseline (speedup 1.0000x reference)
import functools

import jax
import jax.numpy as jnp
from jax import lax
from jax.experimental import pallas as pl
from jax.experimental.pallas import tpu as pltpu
from jax.experimental.pallas import tpu_sc as plsc

N_NODES = 10000
N_EDGES = 320000
D = 128

NUM_CORES = 2
NUM_SUBCORES = 16
NUM_TILES = NUM_CORES * NUM_SUBCORES
EDGES_PER_TILE = N_EDGES // NUM_TILES
CHUNK = 112
CHUNKS_PER_TILE = 90
SUPER = 6
NSUPER = CHUNKS_PER_TILE // SUPER
PAD_EDGES = CHUNKS_PER_TILE * CHUNK - EDGES_PER_TILE
N_ACC = 10240
STRIPE = N_ACC // NUM_SUBCORES


def _make_sc_aggregate(with_counts: bool):
    mesh = plsc.VectorSubcoreMesh(
        core_axis_name="c", subcore_axis_name="s", num_cores=NUM_CORES)

    out_type = [jax.ShapeDtypeStruct((NUM_CORES, N_ACC, D), jnp.float32)]
    scratch = [
        pltpu.VMEM((2, SUPER, CHUNK), jnp.int32),
        pltpu.VMEM((2, SUPER, CHUNK), jnp.int32),
        pltpu.VMEM((CHUNK, D), jnp.float32),
        pltpu.VMEM((CHUNK, D), jnp.float32),
        pltpu.VMEM((CHUNK, D), jnp.float32),
        pltpu.VMEM_SHARED((N_ACC, D), jnp.float32),
        pltpu.SemaphoreType.DMA,
        pltpu.SemaphoreType.DMA,
        pltpu.SemaphoreType.DMA,
        pltpu.SemaphoreType.DMA,
        pltpu.SemaphoreType.DMA,
        pltpu.SemaphoreType.DMA,
        pltpu.SemaphoreType.DMA,
    ]
    if with_counts:
        out_type.append(jax.ShapeDtypeStruct((NUM_CORES, N_ACC), jnp.float32))
        scratch += [
            pltpu.VMEM((CHUNK,), jnp.float32),
            pltpu.VMEM((128,), jnp.float32),
            pltpu.VMEM_SHARED((N_ACC,), jnp.float32),
        ]

    def body(src_hbm, dst_hbm, x_hbm, *rest):
        if with_counts:
            (psum_out, cnt_out, src_v, dst_v, rows0, rows1, rows2, accum,
             semg0, semg1, semg2, sems0, sems1, sems2, semi,
             ones_v, zbuf_v, cnts) = rest
        else:
            (psum_out, src_v, dst_v, rows0, rows1, rows2, accum,
             semg0, semg1, semg2, sems0, sems1, sems2, semi) = rest
            cnt_out = ones_v = zbuf_v = cnts = None
        rows = (rows0, rows1, rows2)
        semg = (semg0, semg1, semg2)
        sems = (sems0, sems1, sems2)

        c = lax.axis_index("c")
        s = lax.axis_index("s")
        wid = s * NUM_CORES + c

        def zrow(r, carry):
            for i in range(D // 16):
                rows0[r, pl.ds(16 * i, 16)] = jnp.zeros((16,), jnp.float32)
            return carry
        lax.fori_loop(0, CHUNK, zrow, 0)
        nfull = STRIPE // CHUNK
        for i in range(nfull):
            pltpu.sync_copy(
                rows0, accum.at[pl.ds(s * STRIPE + i * CHUNK, CHUNK)])
        rem = STRIPE - nfull * CHUNK
        if rem:
            pltpu.sync_copy(
                rows0.at[pl.ds(0, rem)],
                accum.at[pl.ds(s * STRIPE + nfull * CHUNK, rem)])
        if with_counts:
            for i in range(CHUNK // 16):
                ones_v[pl.ds(16 * i, 16)] = jnp.ones((16,), jnp.float32)
            for i in range(128 // 16):
                zbuf_v[pl.ds(16 * i, 16)] = jnp.zeros((16,), jnp.float32)
            for i in range(STRIPE // 128):
                pltpu.sync_copy(
                    zbuf_v, cnts.at[pl.ds(s * STRIPE + i * 128, 128)])

        pltpu.sync_copy(src_hbm.at[wid * NSUPER], src_v.at[0])
        pltpu.sync_copy(dst_hbm.at[wid * NSUPER], dst_v.at[0])

        plsc.subcore_barrier()

        pltpu.async_copy(x_hbm.at[src_v.at[0, 0]], rows0, semg0)
        pltpu.async_copy(x_hbm.at[src_v.at[0, 1]], rows1, semg1)

        def super_body(t, carry):
            slot = lax.rem(t, 2)
            nslot = lax.rem(t + 1, 2)
            not_last = t < NSUPER - 1

            @pl.when(t > 0)
            def _():
                pltpu.make_async_copy(
                    rows[2], accum.at[dst_v.at[0, 0]], sems[2]).wait()

            @pl.when(not_last)
            def _():
                pltpu.async_copy(
                    src_hbm.at[wid * NSUPER + t + 1], src_v.at[nslot], semi)
                pltpu.async_copy(
                    dst_hbm.at[wid * NSUPER + t + 1], dst_v.at[nslot], semi)

            for k in range(SUPER):
                b = k % 3
                pltpu.make_async_copy(
                    x_hbm.at[src_v.at[slot, k]], rows[b], semg[b]).wait()
                pltpu.async_copy(rows[b], accum.at[dst_v.at[slot, k]],
                                 sems[b], add=True)
                if with_counts:
                    pltpu.sync_copy(ones_v, cnts.at[dst_v.at[slot, k]],
                                    add=True)
                if k == SUPER - 3:
                    @pl.when(not_last)
                    def _():
                        pltpu.make_async_copy(
                            src_hbm.at[wid * NSUPER + t + 1],
                            src_v.at[nslot], semi).wait()
                        pltpu.make_async_copy(
                            dst_hbm.at[wid * NSUPER + t + 1],
                            dst_v.at[nslot], semi).wait()
                if k >= 1:
                    bp = (k - 1) % 3
                    pltpu.make_async_copy(
                        rows[bp], accum.at[dst_v.at[slot, k - 1]],
                        sems[bp]).wait()
                bn = (k + 2) % 3
                if k < SUPER - 2:
                    pltpu.async_copy(
                        x_hbm.at[src_v.at[slot, k + 2]], rows[bn], semg[bn])
                else:
                    @pl.when(not_last)
                    def _():
                        pltpu.async_copy(
                            x_hbm.at[src_v.at[nslot, k - (SUPER - 2)]],
                            rows[bn], semg[bn])
            return carry

        lax.fori_loop(0, NSUPER, super_body, 0)
        pltpu.make_async_copy(
            rows[2], accum.at[dst_v.at[0, 0]], sems[2]).wait()

        plsc.subcore_barrier()

        pltpu.sync_copy(accum.at[pl.ds(s * STRIPE, STRIPE)],
                        psum_out.at[c, pl.ds(s * STRIPE, STRIPE)])
        if with_counts:
            pltpu.sync_copy(cnts.at[pl.ds(s * STRIPE, STRIPE)],
                            cnt_out.at[c, pl.ds(s * STRIPE, STRIPE)])

    return pl.kernel(body, out_type=out_type, mesh=mesh,
                     scratch_types=scratch)


_agg_with_counts = _make_sc_aggregate(True)
_agg_no_counts = _make_sc_aggregate(False)


def _tc_xr_body(xref, wrref, bref, oref):
    oref[...] = (jnp.dot(xref[...], wrref[...],
                         preferred_element_type=jnp.float32)
                 + bref[...][None, :])


def _tc_xr(x, wr, b):
    R = 2000
    return pl.pallas_call(
        _tc_xr_body,
        grid=(N_NODES // R,),
        in_specs=[
            pl.BlockSpec((R, D), lambda j: (j, 0)),
            pl.BlockSpec((D, D), lambda j: (0, 0)),
            pl.BlockSpec((D,), lambda j: (0,)),
        ],
        out_specs=pl.BlockSpec((R, D), lambda j: (j, 0)),
        out_shape=jax.ShapeDtypeStruct((N_NODES, D), jnp.float32),
    )(x, wr, b)


def _tc_combine_body(relu, pref, cref, xrref, wlref, oref):
    csum = cref[0] + cref[1]
    inv = 1.0 / jnp.maximum(csum, 1.0)
    mean = (pref[0] + pref[1]) * inv
    y = (jnp.dot(mean, wlref[...], preferred_element_type=jnp.float32)
         + xrref[...])
    oref[...] = jnp.maximum(y, 0.0) if relu else y


def _tc_combine(psum, cnt3, xr, wl, relu):
    R = 2000
    return pl.pallas_call(
        functools.partial(_tc_combine_body, relu),
        grid=(N_NODES // R,),
        in_specs=[
            pl.BlockSpec((NUM_CORES, R, D), lambda j: (0, j, 0)),
            pl.BlockSpec((NUM_CORES, R, 1), lambda j: (0, j, 0)),
            pl.BlockSpec((R, D), lambda j: (j, 0)),
            pl.BlockSpec((D, D), lambda j: (0, 0)),
        ],
        out_specs=pl.BlockSpec((R, D), lambda j: (j, 0)),
        out_shape=jax.ShapeDtypeStruct((N_NODES, D), jnp.float32),
    )(psum, cnt3, xr, wl)


def _pad_idx_body(sref, dref, soref, doref):
    pad_ar = lax.broadcasted_iota(jnp.int32, (NUM_TILES, PAD_EDGES), 1)
    soref[...] = jnp.concatenate([sref[...], pad_ar % 16], axis=1)
    doref[...] = jnp.concatenate([dref[...], N_NODES + pad_ar], axis=1)


def _pad_idx(prop_edge_index):
    npad = CHUNKS_PER_TILE * CHUNK
    s2 = prop_edge_index[0].reshape(NUM_TILES, EDGES_PER_TILE)
    d2 = prop_edge_index[1].reshape(NUM_TILES, EDGES_PER_TILE)
    src_p, dst_p = pl.pallas_call(
        _pad_idx_body,
        out_shape=[
            jax.ShapeDtypeStruct((NUM_TILES, npad), jnp.int32),
            jax.ShapeDtypeStruct((NUM_TILES, npad), jnp.int32),
        ],
    )(s2, d2)
    return (src_p.reshape(NUM_TILES * NSUPER, SUPER, CHUNK),
            dst_p.reshape(NUM_TILES * NSUPER, SUPER, CHUNK))


def kernel(prop_edge_index, emb, Wl1, Wr1, b1, Wl2, Wr2, b2):
    src_p, dst_p = _pad_idx(prop_edge_index)

    xr1 = _tc_xr(emb, Wr1, b1)
    psum1, cnt = _agg_with_counts(src_p, dst_p, emb)
    cnt3 = cnt.reshape(NUM_CORES, N_ACC, 1)
    h1 = _tc_combine(psum1, cnt3, xr1, Wl1, relu=True)
    xr2 = _tc_xr(h1, Wr2, b2)
    (psum2,) = _agg_no_counts(src_p, dst_p, h1)
    return _tc_combine(psum2, cnt3, xr2, Wl2, relu=False)


# speedup vs baseline: 1.0407x; 1.0407x over previous
import functools

import jax
import jax.numpy as jnp
from jax import lax
from jax.experimental import pallas as pl
from jax.experimental.pallas import tpu as pltpu
from jax.experimental.pallas import tpu_sc as plsc

N_NODES = 10000
N_EDGES = 320000
D = 128

NUM_CORES = 2
NUM_SUBCORES = 16
NUM_TILES = NUM_CORES * NUM_SUBCORES
EDGES_PER_TILE = N_EDGES // NUM_TILES
CHUNK = 112
CHUNKS_PER_TILE = 90
SUPER = 6
NSUPER = CHUNKS_PER_TILE // SUPER
PAD_EDGES = CHUNKS_PER_TILE * CHUNK - EDGES_PER_TILE
N_ACC = 10240
STRIPE = N_ACC // NUM_SUBCORES


def _make_sc_aggregate(with_counts: bool):
    mesh = plsc.VectorSubcoreMesh(
        core_axis_name="c", subcore_axis_name="s", num_cores=NUM_CORES)

    out_type = [jax.ShapeDtypeStruct((NUM_CORES, N_ACC, D), jnp.float32)]
    scratch = [
        pltpu.VMEM((2, SUPER, CHUNK), jnp.int32),
        pltpu.VMEM((2, SUPER, CHUNK), jnp.int32),
        pltpu.VMEM((CHUNK, D), jnp.float32),
        pltpu.VMEM((CHUNK, D), jnp.float32),
        pltpu.VMEM((CHUNK, D), jnp.float32),
        pltpu.VMEM_SHARED((N_ACC, D), jnp.float32),
        pltpu.SemaphoreType.DMA,
        pltpu.SemaphoreType.DMA,
        pltpu.SemaphoreType.DMA,
        pltpu.SemaphoreType.DMA,
        pltpu.SemaphoreType.DMA,
        pltpu.SemaphoreType.DMA,
        pltpu.SemaphoreType.DMA,
    ]
    if with_counts:
        out_type.append(jax.ShapeDtypeStruct((NUM_CORES, N_ACC), jnp.float32))
        scratch += [
            pltpu.VMEM((CHUNK,), jnp.float32),
            pltpu.VMEM((128,), jnp.float32),
            pltpu.VMEM_SHARED((N_ACC,), jnp.float32),
        ]

    def body(idx_hbm, x_hbm, *rest):
        if with_counts:
            (psum_out, cnt_out, src_v, dst_v, rows0, rows1, rows2, accum,
             semg0, semg1, semg2, sems0, sems1, sems2, semi,
             ones_v, zbuf_v, cnts) = rest
        else:
            (psum_out, src_v, dst_v, rows0, rows1, rows2, accum,
             semg0, semg1, semg2, sems0, sems1, sems2, semi) = rest
            cnt_out = ones_v = zbuf_v = cnts = None
        rows = (rows0, rows1, rows2)
        semg = (semg0, semg1, semg2)
        sems = (sems0, sems1, sems2)

        c = lax.axis_index("c")
        s = lax.axis_index("s")
        wid = s * NUM_CORES + c

        def zrow(r, carry):
            for i in range(D // 16):
                rows0[r, pl.ds(16 * i, 16)] = jnp.zeros((16,), jnp.float32)
            return carry
        lax.fori_loop(0, CHUNK, zrow, 0)
        nfull = STRIPE // CHUNK
        for i in range(nfull):
            pltpu.sync_copy(
                rows0, accum.at[pl.ds(s * STRIPE + i * CHUNK, CHUNK)])
        rem = STRIPE - nfull * CHUNK
        if rem:
            pltpu.sync_copy(
                rows0.at[pl.ds(0, rem)],
                accum.at[pl.ds(s * STRIPE + nfull * CHUNK, rem)])
        if with_counts:
            for i in range(CHUNK // 16):
                ones_v[pl.ds(16 * i, 16)] = jnp.ones((16,), jnp.float32)
            for i in range(128 // 16):
                zbuf_v[pl.ds(16 * i, 16)] = jnp.zeros((16,), jnp.float32)
            for i in range(STRIPE // 128):
                pltpu.sync_copy(
                    zbuf_v, cnts.at[pl.ds(s * STRIPE + i * 128, 128)])

        soff = wid * NSUPER
        doff = NUM_TILES * NSUPER + wid * NSUPER
        pltpu.sync_copy(idx_hbm.at[soff], src_v.at[0])
        pltpu.sync_copy(idx_hbm.at[doff], dst_v.at[0])

        plsc.subcore_barrier()

        pltpu.async_copy(x_hbm.at[src_v.at[0, 0]], rows0, semg0)
        pltpu.async_copy(x_hbm.at[src_v.at[0, 1]], rows1, semg1)

        def super_body(t, carry):
            slot = lax.rem(t, 2)
            nslot = lax.rem(t + 1, 2)
            not_last = t < NSUPER - 1

            @pl.when(t > 0)
            def _():
                pltpu.make_async_copy(
                    rows[2], accum.at[dst_v.at[0, 0]], sems[2]).wait()

            @pl.when(not_last)
            def _():
                pltpu.async_copy(
                    idx_hbm.at[soff + t + 1], src_v.at[nslot], semi)
                pltpu.async_copy(
                    idx_hbm.at[doff + t + 1], dst_v.at[nslot], semi)

            for k in range(SUPER):
                b = k % 3
                pltpu.make_async_copy(
                    x_hbm.at[src_v.at[slot, k]], rows[b], semg[b]).wait()
                pltpu.async_copy(rows[b], accum.at[dst_v.at[slot, k]],
                                 sems[b], add=True)
                if with_counts:
                    pltpu.sync_copy(ones_v, cnts.at[dst_v.at[slot, k]],
                                    add=True)
                if k == SUPER - 3:
                    @pl.when(not_last)
                    def _():
                        pltpu.make_async_copy(
                            idx_hbm.at[soff + t + 1],
                            src_v.at[nslot], semi).wait()
                        pltpu.make_async_copy(
                            idx_hbm.at[doff + t + 1],
                            dst_v.at[nslot], semi).wait()
                if k >= 1:
                    bp = (k - 1) % 3
                    pltpu.make_async_copy(
                        rows[bp], accum.at[dst_v.at[slot, k - 1]],
                        sems[bp]).wait()
                bn = (k + 2) % 3
                if k < SUPER - 2:
                    pltpu.async_copy(
                        x_hbm.at[src_v.at[slot, k + 2]], rows[bn], semg[bn])
                else:
                    @pl.when(not_last)
                    def _():
                        pltpu.async_copy(
                            x_hbm.at[src_v.at[nslot, k - (SUPER - 2)]],
                            rows[bn], semg[bn])
            return carry

        lax.fori_loop(0, NSUPER, super_body, 0)
        pltpu.make_async_copy(
            rows[2], accum.at[dst_v.at[0, 0]], sems[2]).wait()

        plsc.subcore_barrier()

        pltpu.sync_copy(accum.at[pl.ds(s * STRIPE, STRIPE)],
                        psum_out.at[c, pl.ds(s * STRIPE, STRIPE)])
        if with_counts:
            pltpu.sync_copy(cnts.at[pl.ds(s * STRIPE, STRIPE)],
                            cnt_out.at[c, pl.ds(s * STRIPE, STRIPE)])

    return pl.kernel(body, out_type=out_type, mesh=mesh,
                     scratch_types=scratch)


_agg_with_counts = _make_sc_aggregate(True)
_agg_no_counts = _make_sc_aggregate(False)


def _tc_xr_body(xref, wrref, bref, oref):
    oref[...] = (jnp.dot(xref[...], wrref[...],
                         preferred_element_type=jnp.float32)
                 + bref[...][None, :])


def _tc_xr(x, wr, b):
    R = 2000
    return pl.pallas_call(
        _tc_xr_body,
        grid=(N_NODES // R,),
        in_specs=[
            pl.BlockSpec((R, D), lambda j: (j, 0)),
            pl.BlockSpec((D, D), lambda j: (0, 0)),
            pl.BlockSpec((D,), lambda j: (0,)),
        ],
        out_specs=pl.BlockSpec((R, D), lambda j: (j, 0)),
        out_shape=jax.ShapeDtypeStruct((N_NODES, D), jnp.float32),
    )(x, wr, b)


def _tc_combine_body(relu, pref, cref, xrref, wlref, oref):
    csum = cref[0] + cref[1]
    inv = 1.0 / jnp.maximum(csum, 1.0)
    mean = (pref[0] + pref[1]) * inv
    y = (jnp.dot(mean, wlref[...], preferred_element_type=jnp.float32)
         + xrref[...])
    oref[...] = jnp.maximum(y, 0.0) if relu else y


def _tc_combine(psum, cnt, xr, wl, relu):
    R = 2000
    return pl.pallas_call(
        functools.partial(_tc_combine_body, relu),
        grid=(N_NODES // R,),
        in_specs=[
            pl.BlockSpec((NUM_CORES, R, D), lambda j: (0, j, 0)),
            pl.BlockSpec((NUM_CORES, R, 1), lambda j: (0, j, 0)),
            pl.BlockSpec((R, D), lambda j: (j, 0)),
            pl.BlockSpec((D, D), lambda j: (0, 0)),
        ],
        out_specs=pl.BlockSpec((R, D), lambda j: (j, 0)),
        out_shape=jax.ShapeDtypeStruct((N_NODES, D), jnp.float32),
    )(psum, cnt, xr, wl)


def _pad_idx_body(eref, oref):
    w = pl.program_id(0)
    pad_ar = lax.broadcasted_iota(jnp.int32, (1, NUM_TILES, PAD_EDGES), 2)
    pad = jnp.where(w == 0, pad_ar % 16, N_NODES + pad_ar)
    oref[...] = jnp.concatenate([eref[...], pad], axis=2)


def _pad_idx(prop_edge_index):
    npad = CHUNKS_PER_TILE * CHUNK
    e3 = prop_edge_index.reshape(2, NUM_TILES, EDGES_PER_TILE)
    out = pl.pallas_call(
        _pad_idx_body,
        grid=(2,),
        in_specs=[pl.BlockSpec((1, NUM_TILES, EDGES_PER_TILE),
                               lambda w: (w, 0, 0))],
        out_specs=pl.BlockSpec((1, NUM_TILES, npad), lambda w: (w, 0, 0)),
        out_shape=jax.ShapeDtypeStruct((2, NUM_TILES, npad), jnp.int32),
    )(e3)
    return out.reshape(2 * NUM_TILES * NSUPER, SUPER, CHUNK)


def kernel(prop_edge_index, emb, Wl1, Wr1, b1, Wl2, Wr2, b2):
    idx_p = _pad_idx(prop_edge_index)

    xr1 = _tc_xr(emb, Wr1, b1)
    psum1, cnt = _agg_with_counts(idx_p, emb)
    cnt3 = cnt.reshape(NUM_CORES, N_ACC, 1)
    h1 = _tc_combine(psum1, cnt3, xr1, Wl1, relu=True)
    xr2 = _tc_xr(h1, Wr2, b2)
    (psum2,) = _agg_no_counts(idx_p, h1)
    return _tc_combine(psum2, cnt3, xr2, Wl2, relu=False)


# speedup vs baseline: 1.0682x; 1.0264x over previous
import functools

import jax
import jax.numpy as jnp
from jax import lax
from jax.experimental import pallas as pl
from jax.experimental.pallas import tpu as pltpu
from jax.experimental.pallas import tpu_sc as plsc

N_NODES = 10000
N_EDGES = 320000
D = 128

NUM_CORES = 2
NUM_SUBCORES = 16
NUM_TILES = NUM_CORES * NUM_SUBCORES
EDGES_PER_TILE = N_EDGES // NUM_TILES
CHUNK = 112
CHUNKS_PER_TILE = 90
SUPER = 6
NSUPER = CHUNKS_PER_TILE // SUPER
PAD_EDGES = CHUNKS_PER_TILE * CHUNK - EDGES_PER_TILE
N_ACC = 10240
STRIPE = N_ACC // NUM_SUBCORES


def _make_sc_aggregate(with_counts: bool):
    mesh = plsc.VectorSubcoreMesh(
        core_axis_name="c", subcore_axis_name="s", num_cores=NUM_CORES)

    out_type = [jax.ShapeDtypeStruct((NUM_CORES, N_ACC, D), jnp.float32)]
    scratch = [
        pltpu.VMEM((2, SUPER, CHUNK), jnp.int32),
        pltpu.VMEM((2, SUPER, CHUNK), jnp.int32),
        pltpu.VMEM((CHUNK, D), jnp.float32),
        pltpu.VMEM((CHUNK, D), jnp.float32),
        pltpu.VMEM((CHUNK, D), jnp.float32),
        pltpu.VMEM_SHARED((N_ACC, D), jnp.float32),
        pltpu.SemaphoreType.DMA,
        pltpu.SemaphoreType.DMA,
        pltpu.SemaphoreType.DMA,
        pltpu.SemaphoreType.DMA,
        pltpu.SemaphoreType.DMA,
        pltpu.SemaphoreType.DMA,
        pltpu.SemaphoreType.DMA,
    ]
    if with_counts:
        out_type.append(jax.ShapeDtypeStruct((NUM_CORES, N_ACC), jnp.float32))
        scratch += [
            pltpu.VMEM((CHUNK,), jnp.float32),
            pltpu.VMEM((128,), jnp.float32),
            pltpu.VMEM_SHARED((N_ACC,), jnp.float32),
        ]

    def body(idx_hbm, x_hbm, *rest):
        if with_counts:
            (psum_out, cnt_out, src_v, dst_v, rows0, rows1, rows2, accum,
             semg0, semg1, semg2, sems0, sems1, sems2, semi,
             ones_v, zbuf_v, cnts) = rest
        else:
            (psum_out, src_v, dst_v, rows0, rows1, rows2, accum,
             semg0, semg1, semg2, sems0, sems1, sems2, semi) = rest
            cnt_out = ones_v = zbuf_v = cnts = None
        rows = (rows0, rows1, rows2)
        semg = (semg0, semg1, semg2)
        sems = (sems0, sems1, sems2)

        c = lax.axis_index("c")
        s = lax.axis_index("s")
        wid = s * NUM_CORES + c

        def zrow(r, carry):
            for i in range(D // 16):
                rows0[r, pl.ds(16 * i, 16)] = jnp.zeros((16,), jnp.float32)
            return carry
        lax.fori_loop(0, CHUNK, zrow, 0)
        nfull = STRIPE // CHUNK
        for i in range(nfull):
            pltpu.sync_copy(
                rows0, accum.at[pl.ds(s * STRIPE + i * CHUNK, CHUNK)])
        rem = STRIPE - nfull * CHUNK
        if rem:
            pltpu.sync_copy(
                rows0.at[pl.ds(0, rem)],
                accum.at[pl.ds(s * STRIPE + nfull * CHUNK, rem)])
        if with_counts:
            for i in range(CHUNK // 16):
                ones_v[pl.ds(16 * i, 16)] = jnp.ones((16,), jnp.float32)
            for i in range(128 // 16):
                zbuf_v[pl.ds(16 * i, 16)] = jnp.zeros((16,), jnp.float32)
            for i in range(STRIPE // 128):
                pltpu.sync_copy(
                    zbuf_v, cnts.at[pl.ds(s * STRIPE + i * 128, 128)])

        soff = wid * NSUPER
        doff = NUM_TILES * NSUPER + wid * NSUPER
        pltpu.sync_copy(idx_hbm.at[soff], src_v.at[0])
        pltpu.sync_copy(idx_hbm.at[doff], dst_v.at[0])

        plsc.subcore_barrier()

        pltpu.async_copy(x_hbm.at[src_v.at[0, 0]], rows0, semg0)
        pltpu.async_copy(x_hbm.at[src_v.at[0, 1]], rows1, semg1)

        def super_body(t, carry):
            slot = lax.rem(t, 2)
            nslot = lax.rem(t + 1, 2)
            not_last = t < NSUPER - 1

            @pl.when(t > 0)
            def _():
                pltpu.make_async_copy(
                    rows[2], accum.at[dst_v.at[0, 0]], sems[2]).wait()

            @pl.when(not_last)
            def _():
                pltpu.async_copy(
                    idx_hbm.at[soff + t + 1], src_v.at[nslot], semi)
                pltpu.async_copy(
                    idx_hbm.at[doff + t + 1], dst_v.at[nslot], semi)

            for k in range(SUPER):
                b = k % 3
                pltpu.make_async_copy(
                    x_hbm.at[src_v.at[slot, k]], rows[b], semg[b]).wait()
                pltpu.async_copy(rows[b], accum.at[dst_v.at[slot, k]],
                                 sems[b], add=True)
                if with_counts:
                    pltpu.sync_copy(ones_v, cnts.at[dst_v.at[slot, k]],
                                    add=True)
                if k == SUPER - 3:
                    @pl.when(not_last)
                    def _():
                        pltpu.make_async_copy(
                            idx_hbm.at[soff + t + 1],
                            src_v.at[nslot], semi).wait()
                        pltpu.make_async_copy(
                            idx_hbm.at[doff + t + 1],
                            dst_v.at[nslot], semi).wait()
                if k >= 1:
                    bp = (k - 1) % 3
                    pltpu.make_async_copy(
                        rows[bp], accum.at[dst_v.at[slot, k - 1]],
                        sems[bp]).wait()
                bn = (k + 2) % 3
                if k < SUPER - 2:
                    pltpu.async_copy(
                        x_hbm.at[src_v.at[slot, k + 2]], rows[bn], semg[bn])
                else:
                    @pl.when(not_last)
                    def _():
                        pltpu.async_copy(
                            x_hbm.at[src_v.at[nslot, k - (SUPER - 2)]],
                            rows[bn], semg[bn])
            return carry

        lax.fori_loop(0, NSUPER, super_body, 0)
        pltpu.make_async_copy(
            rows[2], accum.at[dst_v.at[0, 0]], sems[2]).wait()

        plsc.subcore_barrier()

        pltpu.sync_copy(accum.at[pl.ds(s * STRIPE, STRIPE)],
                        psum_out.at[c, pl.ds(s * STRIPE, STRIPE)])
        if with_counts:
            pltpu.sync_copy(cnts.at[pl.ds(s * STRIPE, STRIPE)],
                            cnt_out.at[c, pl.ds(s * STRIPE, STRIPE)])

    return pl.kernel(body, out_type=out_type, mesh=mesh,
                     scratch_types=scratch)


_agg_with_counts = _make_sc_aggregate(True)
_agg_no_counts = _make_sc_aggregate(False)


def _tc_xr_body(xref, wrref, bref, oref):
    oref[...] = (jnp.dot(xref[...], wrref[...],
                         preferred_element_type=jnp.float32)
                 + bref[...][None, :])


def _tc_xr(x, wr, b):
    R = 2000
    return pl.pallas_call(
        _tc_xr_body,
        grid=(N_NODES // R,),
        in_specs=[
            pl.BlockSpec((R, D), lambda j: (j, 0)),
            pl.BlockSpec((D, D), lambda j: (0, 0)),
            pl.BlockSpec((D,), lambda j: (0,)),
        ],
        out_specs=pl.BlockSpec((R, D), lambda j: (j, 0)),
        out_shape=jax.ShapeDtypeStruct((N_NODES, D), jnp.float32),
    )(x, wr, b)


def _tc_combine_body(relu, pref, cref, xrref, wlref, oref):
    csum = cref[0] + cref[1]
    inv = 1.0 / jnp.maximum(csum, 1.0)
    mean = (pref[0] + pref[1]) * inv
    y = (jnp.dot(mean, wlref[...], preferred_element_type=jnp.float32)
         + xrref[...])
    oref[...] = jnp.maximum(y, 0.0) if relu else y


def _tc_combine(psum, cnt, xr, wl, relu):
    R = 2000
    return pl.pallas_call(
        functools.partial(_tc_combine_body, relu),
        grid=(N_NODES // R,),
        in_specs=[
            pl.BlockSpec((NUM_CORES, R, D), lambda j: (0, j, 0)),
            pl.BlockSpec((NUM_CORES, R, 1), lambda j: (0, j, 0)),
            pl.BlockSpec((R, D), lambda j: (j, 0)),
            pl.BlockSpec((D, D), lambda j: (0, 0)),
        ],
        out_specs=pl.BlockSpec((R, D), lambda j: (j, 0)),
        out_shape=jax.ShapeDtypeStruct((N_NODES, D), jnp.float32),
    )(psum, cnt, xr, wl)


def _pad_idx(prop_edge_index):
    pad_ar = jnp.broadcast_to(
        jnp.arange(PAD_EDGES, dtype=jnp.int32)[None, None],
        (2, NUM_TILES, PAD_EDGES))
    pad = jnp.where(jnp.arange(2, dtype=jnp.int32)[:, None, None] == 0,
                    pad_ar % 16, N_NODES + pad_ar)
    e3 = prop_edge_index.reshape(2, NUM_TILES, EDGES_PER_TILE)
    out = jnp.concatenate([e3, pad], axis=2)
    return out.reshape(2 * NUM_TILES * NSUPER, SUPER, CHUNK)


def kernel(prop_edge_index, emb, Wl1, Wr1, b1, Wl2, Wr2, b2):
    idx_p = _pad_idx(prop_edge_index)

    xr1 = _tc_xr(emb, Wr1, b1)
    psum1, cnt = _agg_with_counts(idx_p, emb)
    cnt3 = cnt.reshape(NUM_CORES, N_ACC, 1)
    h1 = _tc_combine(psum1, cnt3, xr1, Wl1, relu=True)
    xr2 = _tc_xr(h1, Wr2, b2)
    (psum2,) = _agg_no_counts(idx_p, h1)
    return _tc_combine(psum2, cnt3, xr2, Wl2, relu=False)
